# Initial kernel scaffold; baseline (speedup 1.0000x reference)
import jax
import jax.numpy as jnp
from jax.experimental import pallas as pl


def kernel(user_indices, visual_tokens):
    raise NotImplementedError("write your pallas kernel here")


# speedup vs baseline: 1.0758x; 1.0758x over previous
import functools

import jax
import jax.numpy as jnp
from jax import lax
from jax.experimental import pallas as pl
from jax.experimental.pallas import tpu as pltpu
from jax.experimental.pallas import tpu_sc as plsc

_NUM_USERS = 100000
_EMBED_DIM = 768
_BATCH = 16384

_NC = 2
_NS = 16
_NW = _NC * _NS
_B_PER_W = _BATCH // _NW
_CHUNK = 128
_N_CHUNKS = _B_PER_W // _CHUNK


def _build_gather():
    mesh = plsc.VectorSubcoreMesh(core_axis_name="c", subcore_axis_name="s")

    @functools.partial(
        pl.kernel,
        mesh=mesh,
        out_type=jax.ShapeDtypeStruct((_BATCH, _EMBED_DIM), jnp.float32),
        scratch_types=[
            pltpu.VMEM((_N_CHUNKS, _CHUNK), jnp.int32),
            pltpu.VMEM((_CHUNK, _EMBED_DIM), jnp.float32),
            pltpu.SemaphoreType.DMA,
        ],
    )
    def gather(idx_hbm, table_hbm, out_hbm, idx_v, rows_v, sem):
        wid = lax.axis_index("s") * _NC + lax.axis_index("c")
        base = wid * _B_PER_W
        pltpu.sync_copy(idx_hbm.at[wid], idx_v)
        for j in range(_N_CHUNKS):
            pltpu.async_copy(table_hbm.at[idx_v.at[j]], rows_v, sem).wait()
            pltpu.sync_copy(rows_v, out_hbm.at[pl.ds(base + j * _CHUNK, _CHUNK)])

    return gather


_gather = _build_gather()


def kernel(user_indices, visual_tokens):
    idx = user_indices.astype(jnp.int32).reshape(_NW, _N_CHUNKS, _CHUNK)
    table = visual_tokens.reshape(_NUM_USERS, _EMBED_DIM)
    out = _gather(idx, table)
    return out.reshape(_BATCH, 1, _EMBED_DIM)


# speedup vs baseline: 1.0813x; 1.0051x over previous
import functools

import jax
import jax.numpy as jnp
from jax import lax
from jax.experimental import pallas as pl
from jax.experimental.pallas import tpu as pltpu
from jax.experimental.pallas import tpu_sc as plsc

_NUM_USERS = 100000
_EMBED_DIM = 768
_BATCH = 16384

_NC = 2
_NS = 16
_NW = _NC * _NS
_B_PER_W = _BATCH // _NW
_CHUNK = 32
_N_CHUNKS = _B_PER_W // _CHUNK
_NBUF = 4
_LEAD = 2


def _build_gather():
    mesh = plsc.VectorSubcoreMesh(core_axis_name="c", subcore_axis_name="s")

    @functools.partial(
        pl.kernel,
        mesh=mesh,
        out_type=jax.ShapeDtypeStruct((_BATCH, _EMBED_DIM), jnp.float32),
        scratch_types=[
            pltpu.VMEM((_N_CHUNKS, _CHUNK), jnp.int32),
            pltpu.VMEM((_NBUF, _CHUNK, _EMBED_DIM), jnp.float32),
            pltpu.SemaphoreType.DMA((_NBUF,)),
            pltpu.SemaphoreType.DMA((_NBUF,)),
        ],
    )
    def gather(idx_hbm, table_hbm, out_hbm, idx_v, rows_v, gsem, osem):
        wid = lax.axis_index("s") * _NC + lax.axis_index("c")
        base = wid * _B_PER_W
        pltpu.sync_copy(idx_hbm.at[wid], idx_v)

        def gstart(j):
            b = j % _NBUF
            return pltpu.async_copy(
                table_hbm.at[idx_v.at[j]], rows_v.at[b], gsem.at[b])

        def ostart(j):
            b = j % _NBUF
            return pltpu.async_copy(
                rows_v.at[b], out_hbm.at[pl.ds(base + j * _CHUNK, _CHUNK)],
                osem.at[b])

        g = [None] * _N_CHUNKS
        o = [None] * _N_CHUNKS
        for j in range(min(_LEAD, _N_CHUNKS)):
            g[j] = gstart(j)
        for j in range(_N_CHUNKS):
            nxt = j + _LEAD
            if nxt < _N_CHUNKS:
                prev = nxt - _NBUF
                if prev >= 0:
                    o[prev].wait()
                g[nxt] = gstart(nxt)
            g[j].wait()
            o[j] = ostart(j)
        for j in range(max(0, _N_CHUNKS - _NBUF + _LEAD), _N_CHUNKS):
            o[j].wait()

    return gather


_gather = _build_gather()


def kernel(user_indices, visual_tokens):
    idx = user_indices.astype(jnp.int32).reshape(_NW, _N_CHUNKS, _CHUNK)
    table = visual_tokens.reshape(_NUM_USERS, _EMBED_DIM)
    out = _gather(idx, table)
    return out.reshape(_BATCH, 1, _EMBED_DIM)


# speedup vs baseline: 6.9051x; 6.3860x over previous
import functools

import jax
import jax.numpy as jnp
from jax import lax
from jax.experimental import pallas as pl
from jax.experimental.pallas import tpu as pltpu
from jax.experimental.pallas import tpu_sc as plsc

_NUM_USERS = 100000
_EMBED_DIM = 768
_BATCH = 16384

_NC = 2
_NS = 16
_NW = _NC * _NS
_B_PER_W = _BATCH // _NW
_CHUNK = 32
_N_CHUNKS = _B_PER_W // _CHUNK
_NBUF = 4
_LEAD = 2


def _build_gather():
    mesh = plsc.VectorSubcoreMesh(core_axis_name="c", subcore_axis_name="s")

    @functools.partial(
        pl.kernel,
        mesh=mesh,
        out_type=jax.ShapeDtypeStruct((_BATCH, 1, _EMBED_DIM), jnp.float32),
        scratch_types=[
            pltpu.VMEM((_N_CHUNKS, _CHUNK), jnp.int32),
            pltpu.VMEM((_NBUF * _CHUNK, 1, _EMBED_DIM), jnp.float32),
            pltpu.SemaphoreType.DMA((_NBUF,)),
            pltpu.SemaphoreType.DMA((_NBUF,)),
        ],
    )
    def gather(idx_hbm, table_hbm, out_hbm, idx_v, rows_v, gsem, osem):
        wid = lax.axis_index("s") * _NC + lax.axis_index("c")
        base = wid * _B_PER_W
        pltpu.sync_copy(idx_hbm.at[wid], idx_v)

        def gstart(j):
            b = j % _NBUF
            return pltpu.async_copy(
                table_hbm.at[idx_v.at[j]],
                rows_v.at[pl.ds(b * _CHUNK, _CHUNK)], gsem.at[b])

        def ostart(j):
            b = j % _NBUF
            return pltpu.async_copy(
                rows_v.at[pl.ds(b * _CHUNK, _CHUNK)],
                out_hbm.at[pl.ds(base + j * _CHUNK, _CHUNK)],
                osem.at[b])

        g = [None] * _N_CHUNKS
        o = [None] * _N_CHUNKS
        for j in range(min(_LEAD, _N_CHUNKS)):
            g[j] = gstart(j)
        for j in range(_N_CHUNKS):
            nxt = j + _LEAD
            if nxt < _N_CHUNKS:
                prev = nxt - _NBUF
                if prev >= 0:
                    o[prev].wait()
                g[nxt] = gstart(nxt)
            g[j].wait()
            o[j] = ostart(j)
        for j in range(max(0, _N_CHUNKS - _NBUF + _LEAD), _N_CHUNKS):
            o[j].wait()

    return gather


_gather = _build_gather()


def kernel(user_indices, visual_tokens):
    idx = user_indices.astype(jnp.int32).reshape(_NW, _N_CHUNKS, _CHUNK)
    return _gather(idx, visual_tokens)


# speedup vs baseline: 6.9695x; 1.0093x over previous
import functools

import jax
import jax.numpy as jnp
from jax import lax
from jax.experimental import pallas as pl
from jax.experimental.pallas import tpu as pltpu
from jax.experimental.pallas import tpu_sc as plsc

_NUM_USERS = 100000
_EMBED_DIM = 768
_BATCH = 16384

_NC = 2
_NS = 16
_NW = _NC * _NS
_B_PER_W = _BATCH // _NW
_CHUNK = 32
_N_CHUNKS = _B_PER_W // _CHUNK
_NBUF = 4
_LEAD = 2


def _build_gather():
    mesh = plsc.VectorSubcoreMesh(core_axis_name="c", subcore_axis_name="s")

    @functools.partial(
        pl.kernel,
        mesh=mesh,
        out_type=jax.ShapeDtypeStruct((_BATCH, 1, _EMBED_DIM), jnp.float32),
        scratch_types=[
            pltpu.VMEM((_B_PER_W,), jnp.int32),
            pltpu.VMEM((_NBUF * _CHUNK, 1, _EMBED_DIM), jnp.float32),
            pltpu.SemaphoreType.DMA((_NBUF,)),
            pltpu.SemaphoreType.DMA((_NBUF,)),
        ],
    )
    def gather(idx_hbm, table_hbm, out_hbm, idx_v, rows_v, gsem, osem):
        wid = lax.axis_index("s") * _NC + lax.axis_index("c")
        base = wid * _B_PER_W
        pltpu.sync_copy(idx_hbm.at[pl.ds(base, _B_PER_W)], idx_v)

        def gstart(j):
            b = j % _NBUF
            return pltpu.async_copy(
                table_hbm.at[idx_v.at[pl.ds(j * _CHUNK, _CHUNK)]],
                rows_v.at[pl.ds(b * _CHUNK, _CHUNK)], gsem.at[b])

        def ostart(j):
            b = j % _NBUF
            return pltpu.async_copy(
                rows_v.at[pl.ds(b * _CHUNK, _CHUNK)],
                out_hbm.at[pl.ds(base + j * _CHUNK, _CHUNK)],
                osem.at[b])

        g = [None] * _N_CHUNKS
        o = [None] * _N_CHUNKS
        for j in range(min(_LEAD, _N_CHUNKS)):
            g[j] = gstart(j)
        for j in range(_N_CHUNKS):
            nxt = j + _LEAD
            if nxt < _N_CHUNKS:
                prev = nxt - _NBUF
                if prev >= 0:
                    o[prev].wait()
                g[nxt] = gstart(nxt)
            g[j].wait()
            o[j] = ostart(j)
        for j in range(max(0, _N_CHUNKS - _NBUF + _LEAD), _N_CHUNKS):
            o[j].wait()

    return gather


_gather = _build_gather()


def kernel(user_indices, visual_tokens):
    return _gather(user_indices.astype(jnp.int32), visual_tokens)
